# baseline (speedup 1.0000x reference)
import jax
import jax.numpy as jnp
from jax.experimental import pallas as pl
from jax.experimental.pallas import tpu as pltpu

PROP_BM = 400
PROJ_BM = 800
NPROJ = 5


def _body(s1_ref, s2_ref, w1_ref, w2_ref, adj_ref, diff_ref, m_ref,
          bb1_ref, bb2_ref, a1_ref, a2_ref, wd_ref, bd_ref,
          h1_ref, h3_ref, h2_ref, h4_ref, out_ref,
          fA_ref, fD_ref, hsA_ref, hsD_ref):
    i = pl.program_id(1)
    n_i = pl.num_programs(1)
    nh = w1_ref.shape[0]
    dn = (((1,), (1,)), ((), ()))

    @pl.when(i < NPROJ)
    def _project():
        s1 = s1_ref[0]
        s2 = s2_ref[0]
        w1 = w1_ref[...]
        w2 = w2_ref[...]
        bf = jnp.bfloat16
        row = i * PROJ_BM
        fA_ref[pl.ds(row, PROJ_BM), :nh] = jax.lax.dot_general(
            s1, w1, dn, preferred_element_type=jnp.float32).astype(bf)
        fA_ref[pl.ds(row, PROJ_BM), nh:] = jax.lax.dot_general(
            s2, w1, dn, preferred_element_type=jnp.float32).astype(bf)
        fD_ref[pl.ds(row, PROJ_BM), :nh] = jax.lax.dot_general(
            s1, w2, dn, preferred_element_type=jnp.float32).astype(bf)
        fD_ref[pl.ds(row, PROJ_BM), nh:] = jax.lax.dot_general(
            s2, w2, dn, preferred_element_type=jnp.float32).astype(bf)

    @pl.when(i >= NPROJ)
    def _propagate():
        bf = jnp.bfloat16
        a = adj_ref[0].astype(bf)
        d = diff_ref[0].astype(bf)
        rA = jnp.dot(a, fA_ref[...],
                     preferred_element_type=jnp.float32) + bb1_ref[...]
        rD = jnp.dot(d, fD_ref[...],
                     preferred_element_type=jnp.float32) + bb2_ref[...]
        a1 = a1_ref[0, 0]
        a2 = a2_ref[0, 0]
        hA = jnp.where(rA > 0, rA, a1 * rA)
        hD = jnp.where(rD > 0, rD, a2 * rD)
        h1_ref[0] = hA[:, :nh]
        h3_ref[0] = hA[:, nh:]
        h2_ref[0] = hD[:, :nh]
        h4_ref[0] = hD[:, nh:]
        row = (i - NPROJ) * PROP_BM
        hsA_ref[pl.ds(row, PROP_BM), :] = hA
        hsD_ref[pl.ds(row, PROP_BM), :] = hD

    @pl.when(i == n_i - 1)
    def _score():
        hA = hsA_ref[...]
        hD = hsD_ref[...]
        h1 = hA[:, :nh]
        h3 = hA[:, nh:]
        h2 = hD[:, :nh]
        h4 = hD[:, nh:]
        m = m_ref[0]
        msum = jnp.sum(m)
        wd = wd_ref[...]
        bd = bd_ref[0, 0]
        rA = jnp.dot(m, hA, preferred_element_type=jnp.float32) / msum
        rD = jnp.dot(m, hD, preferred_element_type=jnp.float32) / msum
        c1 = jax.nn.sigmoid(rA[:, :nh])
        c2 = jax.nn.sigmoid(rD[:, :nh])
        t1 = jnp.dot(h1, wd, preferred_element_type=jnp.float32)
        t2 = jnp.dot(h2, wd, preferred_element_type=jnp.float32)
        t3 = jnp.dot(h3, wd, preferred_element_type=jnp.float32)
        t4 = jnp.dot(h4, wd, preferred_element_type=jnp.float32)
        out_ref[0, :, 0:1] = jnp.sum(t2 * c1, axis=1, keepdims=True) + bd
        out_ref[0, :, 1:2] = jnp.sum(t1 * c2, axis=1, keepdims=True) + bd
        out_ref[0, :, 2:3] = jnp.sum(t4 * c1, axis=1, keepdims=True) + bd
        out_ref[0, :, 3:4] = jnp.sum(t3 * c2, axis=1, keepdims=True) + bd


def kernel(seq1, seq2, adj, diff, sparse, msk, samp_bias1, samp_bias2,
           W1, b1, a1, W2, b2, a2, Wd, bd):
    f32 = jnp.float32
    B, N, n_in = seq1.shape
    nh = W1.shape[0]
    ni = N // PROP_BM

    bb1 = jnp.concatenate([b1, b1]).reshape(1, 2 * nh)
    bb2 = jnp.concatenate([b2, b2]).reshape(1, 2 * nh)
    a1s = jnp.reshape(a1, (1, 1)).astype(f32)
    a2s = jnp.reshape(a2, (1, 1)).astype(f32)
    bds = jnp.reshape(bd, (1, 1)).astype(f32)
    m3 = msk.reshape(B, 1, N)

    def seq_idx(b, i):
        return (b, jnp.minimum(i, NPROJ - 1), 0)

    def prop_idx(b, i):
        return (b, jnp.maximum(i - NPROJ, 0), 0)

    h1, h3, h2, h4, scores = pl.pallas_call(
        _body,
        grid=(B, NPROJ + ni),
        in_specs=[
            pl.BlockSpec((1, PROJ_BM, n_in), seq_idx),
            pl.BlockSpec((1, PROJ_BM, n_in), seq_idx),
            pl.BlockSpec((nh, n_in), lambda b, i: (0, 0)),
            pl.BlockSpec((nh, n_in), lambda b, i: (0, 0)),
            pl.BlockSpec((1, PROP_BM, N), prop_idx),
            pl.BlockSpec((1, PROP_BM, N), prop_idx),
            pl.BlockSpec((1, 1, N), lambda b, i: (b, 0, 0)),
            pl.BlockSpec((1, 2 * nh), lambda b, i: (0, 0)),
            pl.BlockSpec((1, 2 * nh), lambda b, i: (0, 0)),
            pl.BlockSpec((1, 1), lambda b, i: (0, 0)),
            pl.BlockSpec((1, 1), lambda b, i: (0, 0)),
            pl.BlockSpec((nh, nh), lambda b, i: (0, 0)),
            pl.BlockSpec((1, 1), lambda b, i: (0, 0)),
        ],
        out_specs=[
            pl.BlockSpec((1, PROP_BM, nh), prop_idx),
            pl.BlockSpec((1, PROP_BM, nh), prop_idx),
            pl.BlockSpec((1, PROP_BM, nh), prop_idx),
            pl.BlockSpec((1, PROP_BM, nh), prop_idx),
            pl.BlockSpec((1, N, 4), lambda b, i: (b, 0, 0)),
        ],
        out_shape=[
            jax.ShapeDtypeStruct((B, N, nh), f32),
            jax.ShapeDtypeStruct((B, N, nh), f32),
            jax.ShapeDtypeStruct((B, N, nh), f32),
            jax.ShapeDtypeStruct((B, N, nh), f32),
            jax.ShapeDtypeStruct((B, N, 4), f32),
        ],
        scratch_shapes=[
            pltpu.VMEM((N, 2 * nh), jnp.bfloat16),
            pltpu.VMEM((N, 2 * nh), jnp.bfloat16),
            pltpu.VMEM((N, 2 * nh), f32),
            pltpu.VMEM((N, 2 * nh), f32),
        ],
        compiler_params=pltpu.CompilerParams(
            dimension_semantics=("arbitrary", "arbitrary")),
    )(seq1, seq2, W1, W2, adj, diff, m3, bb1, bb2, a1s, a2s, Wd[0], bds)

    logits = scores.transpose(0, 2, 1).reshape(B, 4 * N)
    return (logits, h1, h2)


# speedup vs baseline: 1.0161x; 1.0161x over previous
import jax
import jax.numpy as jnp
from jax.experimental import pallas as pl
from jax.experimental.pallas import tpu as pltpu

PROP_BM = 400
PROJ_BM = 800
NPROJ = 5


def _body(s1_ref, s2_ref, w1_ref, w2_ref, adj_ref, diff_ref, m_ref,
          bb1_ref, bb2_ref, a1_ref, a2_ref, wd_ref, bd_ref,
          h1_ref, h2_ref, out_ref,
          fA_ref, fD_ref, hsA_ref, hsD_ref):
    i = pl.program_id(1)
    n_i = pl.num_programs(1)
    nh = w1_ref.shape[0]
    dn = (((1,), (1,)), ((), ()))

    @pl.when(i < NPROJ)
    def _project():
        s1 = s1_ref[0]
        s2 = s2_ref[0]
        w1 = w1_ref[...]
        w2 = w2_ref[...]
        bf = jnp.bfloat16
        row = i * PROJ_BM
        fA_ref[pl.ds(row, PROJ_BM), :nh] = jax.lax.dot_general(
            s1, w1, dn, preferred_element_type=jnp.float32).astype(bf)
        fA_ref[pl.ds(row, PROJ_BM), nh:] = jax.lax.dot_general(
            s2, w1, dn, preferred_element_type=jnp.float32).astype(bf)
        fD_ref[pl.ds(row, PROJ_BM), :nh] = jax.lax.dot_general(
            s1, w2, dn, preferred_element_type=jnp.float32).astype(bf)
        fD_ref[pl.ds(row, PROJ_BM), nh:] = jax.lax.dot_general(
            s2, w2, dn, preferred_element_type=jnp.float32).astype(bf)

    @pl.when(i >= NPROJ)
    def _propagate():
        bf = jnp.bfloat16
        a = adj_ref[0].astype(bf)
        d = diff_ref[0].astype(bf)
        rA = jnp.dot(a, fA_ref[...],
                     preferred_element_type=jnp.float32) + bb1_ref[...]
        rD = jnp.dot(d, fD_ref[...],
                     preferred_element_type=jnp.float32) + bb2_ref[...]
        a1 = a1_ref[0, 0]
        a2 = a2_ref[0, 0]
        hA = jnp.where(rA > 0, rA, a1 * rA)
        hD = jnp.where(rD > 0, rD, a2 * rD)
        h1_ref[0] = hA[:, :nh]
        h2_ref[0] = hD[:, :nh]
        row = (i - NPROJ) * PROP_BM
        hsA_ref[pl.ds(row, PROP_BM), :] = hA
        hsD_ref[pl.ds(row, PROP_BM), :] = hD

    @pl.when(i == n_i - 1)
    def _score():
        hA = hsA_ref[...]
        hD = hsD_ref[...]
        h1 = hA[:, :nh]
        h3 = hA[:, nh:]
        h2 = hD[:, :nh]
        h4 = hD[:, nh:]
        m = m_ref[0]
        msum = jnp.sum(m)
        wd = wd_ref[...]
        bd = bd_ref[0, 0]
        rA = jnp.dot(m, hA, preferred_element_type=jnp.float32) / msum
        rD = jnp.dot(m, hD, preferred_element_type=jnp.float32) / msum
        c1 = jax.nn.sigmoid(rA[:, :nh])
        c2 = jax.nn.sigmoid(rD[:, :nh])
        t1 = jnp.dot(h1, wd, preferred_element_type=jnp.float32)
        t2 = jnp.dot(h2, wd, preferred_element_type=jnp.float32)
        t3 = jnp.dot(h3, wd, preferred_element_type=jnp.float32)
        t4 = jnp.dot(h4, wd, preferred_element_type=jnp.float32)
        out_ref[0, :, 0:1] = jnp.sum(t2 * c1, axis=1, keepdims=True) + bd
        out_ref[0, :, 1:2] = jnp.sum(t1 * c2, axis=1, keepdims=True) + bd
        out_ref[0, :, 2:3] = jnp.sum(t4 * c1, axis=1, keepdims=True) + bd
        out_ref[0, :, 3:4] = jnp.sum(t3 * c2, axis=1, keepdims=True) + bd


def kernel(seq1, seq2, adj, diff, sparse, msk, samp_bias1, samp_bias2,
           W1, b1, a1, W2, b2, a2, Wd, bd):
    f32 = jnp.float32
    B, N, n_in = seq1.shape
    nh = W1.shape[0]
    ni = N // PROP_BM

    bb1 = jnp.concatenate([b1, b1]).reshape(1, 2 * nh)
    bb2 = jnp.concatenate([b2, b2]).reshape(1, 2 * nh)
    a1s = jnp.reshape(a1, (1, 1)).astype(f32)
    a2s = jnp.reshape(a2, (1, 1)).astype(f32)
    bds = jnp.reshape(bd, (1, 1)).astype(f32)
    m3 = msk.reshape(B, 1, N)

    def seq_idx(b, i):
        return (b, jnp.minimum(i, NPROJ - 1), 0)

    def prop_idx(b, i):
        return (b, jnp.maximum(i - NPROJ, 0), 0)

    h1, h2, scores = pl.pallas_call(
        _body,
        grid=(B, NPROJ + ni),
        in_specs=[
            pl.BlockSpec((1, PROJ_BM, n_in), seq_idx),
            pl.BlockSpec((1, PROJ_BM, n_in), seq_idx),
            pl.BlockSpec((nh, n_in), lambda b, i: (0, 0)),
            pl.BlockSpec((nh, n_in), lambda b, i: (0, 0)),
            pl.BlockSpec((1, PROP_BM, N), prop_idx),
            pl.BlockSpec((1, PROP_BM, N), prop_idx),
            pl.BlockSpec((1, 1, N), lambda b, i: (b, 0, 0)),
            pl.BlockSpec((1, 2 * nh), lambda b, i: (0, 0)),
            pl.BlockSpec((1, 2 * nh), lambda b, i: (0, 0)),
            pl.BlockSpec((1, 1), lambda b, i: (0, 0)),
            pl.BlockSpec((1, 1), lambda b, i: (0, 0)),
            pl.BlockSpec((nh, nh), lambda b, i: (0, 0)),
            pl.BlockSpec((1, 1), lambda b, i: (0, 0)),
        ],
        out_specs=[
            pl.BlockSpec((1, PROP_BM, nh), prop_idx),
            pl.BlockSpec((1, PROP_BM, nh), prop_idx),
            pl.BlockSpec((1, N, 4), lambda b, i: (b, 0, 0)),
        ],
        out_shape=[
            jax.ShapeDtypeStruct((B, N, nh), f32),
            jax.ShapeDtypeStruct((B, N, nh), f32),
            jax.ShapeDtypeStruct((B, N, 4), f32),
        ],
        scratch_shapes=[
            pltpu.VMEM((N, 2 * nh), jnp.bfloat16),
            pltpu.VMEM((N, 2 * nh), jnp.bfloat16),
            pltpu.VMEM((N, 2 * nh), f32),
            pltpu.VMEM((N, 2 * nh), f32),
        ],
        compiler_params=pltpu.CompilerParams(
            dimension_semantics=("arbitrary", "arbitrary")),
    )(seq1, seq2, W1, W2, adj, diff, m3, bb1, bb2, a1s, a2s, Wd[0], bds)

    logits = scores.transpose(0, 2, 1).reshape(B, 4 * N)
    return (logits, h1, h2)
